# baseline (speedup 1.0000x reference)
import functools

import jax
import jax.numpy as jnp
from jax import lax
from jax.experimental import pallas as pl
from jax.experimental.pallas import tpu as pltpu
from jax.experimental.pallas import tpu_sc as plsc

NUM_DATA = 100000
NUM_TRUNC = 10000
D = 128
BLK = 128
NT = 16
NC = 2
PAD_DST = 1 << 30


def _make_spmm(n_edges_pad, out_rows, chunk_rows, acc_rows, n_chunks_per_sc):
    stripe = n_edges_pad // NT
    nblk = stripe // BLK
    assert stripe % BLK == 0
    assert chunk_rows % (NT * 32) == 0 and acc_rows % (NT * 32) == 0
    assert acc_rows >= chunk_rows + NT
    assert chunk_rows * n_chunks_per_sc * NC == out_rows
    mesh = plsc.VectorSubcoreMesh(core_axis_name="c", subcore_axis_name="s")

    @functools.partial(
        pl.kernel,
        out_type=jax.ShapeDtypeStruct((out_rows, D), jnp.float32),
        mesh=mesh,
        scratch_types=[
            pltpu.VMEM((BLK,), jnp.int32),
            pltpu.VMEM((BLK,), jnp.int32),
            pltpu.VMEM((BLK,), jnp.float32),
            pltpu.VMEM((BLK, D), jnp.float32),
            pltpu.VMEM((32, D), jnp.float32),
            pltpu.VMEM_SHARED((acc_rows, D), jnp.float32),
            pltpu.SemaphoreType.DMA,
        ],
    )
    def spmm(src_hbm, dst_hbm, w_hbm, table_hbm, out_hbm,
             src_v, ldst_v, w_v, rows_v, zero_v, acc, sem):
        cid = lax.axis_index("c")
        tid = lax.axis_index("s")
        dump_row = chunk_rows + tid

        def zrow(r, _):
            for v in range(D // 16):
                zero_v[r, pl.ds(v * 16, 16)] = jnp.zeros((16,), jnp.float32)
            return 0
        lax.fori_loop(0, 32, zrow, 0)

        for c in range(n_chunks_per_sc):
            cb = (cid * n_chunks_per_sc + c) * chunk_rows

            for z in range(acc_rows // (NT * 32)):
                r0 = tid * (acc_rows // NT) + z * 32
                pltpu.sync_copy(zero_v, acc.at[pl.ds(r0, 32)])
            plsc.subcore_barrier()

            def block(b, _):
                eb = tid * stripe + b * BLK
                pltpu.sync_copy(src_hbm.at[pl.ds(eb, BLK)], src_v)
                pltpu.sync_copy(dst_hbm.at[pl.ds(eb, BLK)], ldst_v)
                pltpu.sync_copy(w_hbm.at[pl.ds(eb, BLK)], w_v)
                for v in range(BLK // 16):
                    sl = pl.ds(v * 16, 16)
                    l16 = ldst_v[sl] - cb
                    inr = (l16 >= 0) & (l16 < chunk_rows)
                    ldst_v[sl] = jnp.where(inr, l16, dump_row)
                    w_v[sl] = jnp.where(inr, w_v[sl], 0.0)
                pltpu.async_copy(table_hbm.at[src_v], rows_v, sem).wait()

                def scale(g, _):
                    w16 = w_v[pl.ds(g * 16, 16)]
                    for l in range(16):
                        w = w16[l]
                        j = g * 16 + l
                        for v in range(D // 16):
                            sl = pl.ds(v * 16, 16)
                            rows_v[j, sl] = rows_v[j, sl] * w
                    return 0
                lax.fori_loop(0, BLK // 16, scale, 0)
                pltpu.sync_copy(rows_v, acc.at[ldst_v], add=True)
                return 0
            lax.fori_loop(0, nblk, block, 0)
            plsc.subcore_barrier()

            rpt = chunk_rows // NT
            for z in range(rpt // 32):
                r0 = tid * rpt + z * 32
                pltpu.sync_copy(acc.at[pl.ds(r0, 32)],
                                out_hbm.at[pl.ds(cb + r0, 32)])
            plsc.subcore_barrier()

    return spmm


def _pad_edges(src, dst, w, n_pad):
    e = src.shape[0]
    pad = n_pad - e
    src = jnp.concatenate([src, jnp.zeros((pad,), jnp.int32)])
    dst = jnp.concatenate([dst, jnp.full((pad,), PAD_DST, jnp.int32)])
    w = jnp.concatenate([w, jnp.zeros((pad,), jnp.float32)])
    return src, dst, w


_E_PAD = 401408

_down_spmm = _make_spmm(_E_PAD, out_rows=10240, chunk_rows=5120,
                        acc_rows=5632, n_chunks_per_sc=1)
_up_spmm = _make_spmm(_E_PAD, out_rows=102400, chunk_rows=12800,
                      acc_rows=13312, n_chunks_per_sc=4)


def kernel(x, down_w, up_w, down_edge_index, up_edge_index):
    xl = x[:, -1]
    b, e, g, f = xl.shape
    x2d = xl.reshape(g, f)
    dsrc, ddst, dw = _pad_edges(down_edge_index[0], down_edge_index[1],
                                down_w, _E_PAD)
    usrc, udst, uw = _pad_edges(up_edge_index[0], up_edge_index[1],
                                up_w, _E_PAD)
    down_table = _down_spmm(dsrc, ddst, dw, x2d)
    up = _up_spmm(usrc, udst, uw, down_table)
    return up[:NUM_DATA].reshape(b, e, NUM_DATA, f)


# speedup vs baseline: 1.5697x; 1.5697x over previous
import functools

import jax
import jax.numpy as jnp
from jax import lax
from jax.experimental import pallas as pl
from jax.experimental.pallas import tpu as pltpu
from jax.experimental.pallas import tpu_sc as plsc

NUM_DATA = 100000
NUM_TRUNC = 10000
D = 128
BLK = 128
NT = 16
NC = 2
PAD_DST = 1 << 30


def _make_spmm(n_edges_pad, out_rows, chunk_rows, acc_rows, n_chunks_per_sc,
               pcap):
    stripe = n_edges_pad // NT
    nblk = stripe // BLK
    cap = pcap * BLK
    assert stripe % BLK == 0
    assert chunk_rows % (NT * 32) == 0 and acc_rows % (NT * 8) == 0
    assert acc_rows >= chunk_rows + NT
    assert chunk_rows * n_chunks_per_sc * NC == out_rows
    mesh = plsc.VectorSubcoreMesh(core_axis_name="c", subcore_axis_name="s")

    @functools.partial(
        pl.kernel,
        out_type=jax.ShapeDtypeStruct((out_rows, D), jnp.float32),
        mesh=mesh,
        scratch_types=[
            pltpu.VMEM((BLK,), jnp.int32),
            pltpu.VMEM((BLK,), jnp.int32),
            pltpu.VMEM((BLK,), jnp.float32),
            pltpu.VMEM((cap + BLK,), jnp.int32),
            pltpu.VMEM((cap + BLK,), jnp.int32),
            pltpu.VMEM((cap + BLK,), jnp.float32),
            pltpu.VMEM((BLK, D), jnp.float32),
            pltpu.VMEM((8, D), jnp.float32),
            pltpu.VMEM_SHARED((acc_rows, D), jnp.float32),
            pltpu.SemaphoreType.DMA,
        ],
        compiler_params=pltpu.CompilerParams(needs_layout_passes=False),
    )
    def spmm(src_hbm, dst_hbm, w_hbm, table_hbm, out_hbm,
             src_v, dst_v, w_v, psrc, pldst, pw, rows_v, zero_v, acc, sem):
        cid = lax.axis_index("c")
        tid = lax.axis_index("s")
        dump_row = chunk_rows + tid

        def zrow(r, _):
            for v in range(D // 16):
                zero_v[r, pl.ds(v * 16, 16)] = jnp.zeros((16,), jnp.float32)
            return 0
        lax.fori_loop(0, 8, zrow, 0)

        def flush_block(b, _):
            for v in range(BLK // 16):
                sl = pl.ds(v * 16, 16)
                src_v[sl] = psrc[pl.ds(b * BLK + v * 16, 16)]
                dst_v[sl] = pldst[pl.ds(b * BLK + v * 16, 16)]
            pltpu.async_copy(table_hbm.at[src_v], rows_v, sem).wait()

            def scale(g, _):
                w16 = pw[pl.ds(b * BLK + g * 16, 16)]
                for l in range(16):
                    w = w16[l]
                    j = g * 16 + l
                    for v in range(D // 16):
                        sl = pl.ds(v * 16, 16)
                        rows_v[j, sl] = rows_v[j, sl] * w
                return 0
            lax.fori_loop(0, BLK // 16, scale, 0)
            pltpu.sync_copy(rows_v, acc.at[dst_v], add=True)
            return 0

        def chunk_body(c, _):
            cb = (cid * n_chunks_per_sc + c) * chunk_rows

            for z in range(acc_rows // (NT * 8)):
                r0 = tid * (acc_rows // NT) + z * 8
                pltpu.sync_copy(zero_v, acc.at[pl.ds(r0, 8)])
            plsc.subcore_barrier()

            def compact(b, pend):
                eb = tid * stripe + b * BLK
                pltpu.sync_copy(src_hbm.at[pl.ds(eb, BLK)], src_v)
                pltpu.sync_copy(dst_hbm.at[pl.ds(eb, BLK)], dst_v)
                pltpu.sync_copy(w_hbm.at[pl.ds(eb, BLK)], w_v)
                for v in range(BLK // 16):
                    sl = pl.ds(v * 16, 16)
                    l16 = dst_v[sl] - cb
                    inr = (l16 >= 0) & (l16 < chunk_rows)
                    plsc.store_compressed(psrc.at[pl.ds(pend, 16)],
                                          src_v[sl], mask=inr)
                    plsc.store_compressed(pldst.at[pl.ds(pend, 16)],
                                          l16, mask=inr)
                    plsc.store_compressed(pw.at[pl.ds(pend, 16)],
                                          w_v[sl], mask=inr)
                    pend = pend + plsc.all_reduce_population_count(inr)[0]

                def overflow(p):
                    nfull = p >> 7
                    lax.fori_loop(0, nfull, flush_block, 0)
                    for v in range(BLK // 16):
                        sl = pl.ds(v * 16, 16)
                        off = pl.ds(nfull * BLK + v * 16, 16)
                        psrc[sl] = psrc[off]
                        pldst[sl] = pldst[off]
                        pw[sl] = pw[off]
                    return p & (BLK - 1)
                return lax.cond(pend >= cap - BLK, overflow,
                                lambda p: p, pend)
            pend = lax.fori_loop(0, nblk, compact, jnp.int32(0))

            for v in range(BLK // 16):
                off = pl.ds(pend + v * 16, 16)
                psrc[off] = jnp.zeros((16,), jnp.int32)
                pldst[off] = jnp.full((16,), dump_row, jnp.int32)
                pw[off] = jnp.zeros((16,), jnp.float32)
            nflush = (pend + BLK - 1) >> 7
            lax.fori_loop(0, nflush, flush_block, 0)
            plsc.subcore_barrier()

            rpt = chunk_rows // NT
            for z in range(rpt // 32):
                r0 = tid * rpt + z * 32
                pltpu.sync_copy(acc.at[pl.ds(r0, 32)],
                                out_hbm.at[pl.ds(cb + r0, 32)])
            plsc.subcore_barrier()
            return 0
        lax.fori_loop(0, n_chunks_per_sc, chunk_body, 0)

    return spmm


def _pad_edges(src, dst, w, n_pad):
    e = src.shape[0]
    pad = n_pad - e
    src = jnp.concatenate([src, jnp.zeros((pad,), jnp.int32)])
    dst = jnp.concatenate([dst, jnp.full((pad,), PAD_DST, jnp.int32)])
    w = jnp.concatenate([w, jnp.zeros((pad,), jnp.float32)])
    return src, dst, w


_E_PAD = 401408

_down_spmm = _make_spmm(_E_PAD, out_rows=10240, chunk_rows=5120,
                        acc_rows=5632, n_chunks_per_sc=1, pcap=168)
_up_spmm = _make_spmm(_E_PAD, out_rows=102400, chunk_rows=10240,
                      acc_rows=10752, n_chunks_per_sc=5, pcap=64)


def kernel(x, down_w, up_w, down_edge_index, up_edge_index):
    xl = x[:, -1]
    b, e, g, f = xl.shape
    x2d = xl.reshape(g, f)
    dsrc, ddst, dw = _pad_edges(down_edge_index[0], down_edge_index[1],
                                down_w, _E_PAD)
    usrc, udst, uw = _pad_edges(up_edge_index[0], up_edge_index[1],
                                up_w, _E_PAD)
    down_table = _down_spmm(dsrc, ddst, dw, x2d)
    up = _up_spmm(usrc, udst, uw, down_table)
    return up[:NUM_DATA].reshape(b, e, NUM_DATA, f)


# speedup vs baseline: 3.2042x; 2.0413x over previous
import functools

import jax
import jax.numpy as jnp
from jax import lax
from jax.experimental import pallas as pl
from jax.experimental.pallas import tpu as pltpu
from jax.experimental.pallas import tpu_sc as plsc

NUM_DATA = 100000
NUM_TRUNC = 10000
D = 128
BLK = 128
SEG = 1024
NT = 16
NC = 2
PAD_DST = 1 << 30


def _make_spmm(n_edges_pad, out_rows, chunk_rows, acc_rows, n_chunks_per_sc,
               pcap):
    stripe = n_edges_pad // NT
    nseg = stripe // SEG
    cap = pcap * BLK
    assert stripe % SEG == 0
    assert chunk_rows % (NT * 32) == 0 and acc_rows % (NT * 8) == 0
    assert acc_rows >= chunk_rows + NT
    assert chunk_rows * n_chunks_per_sc * NC == out_rows
    mesh = plsc.VectorSubcoreMesh(core_axis_name="c", subcore_axis_name="s")

    @functools.partial(
        pl.kernel,
        out_type=jax.ShapeDtypeStruct((out_rows, D), jnp.float32),
        mesh=mesh,
        scratch_types=[
            pltpu.VMEM((2, 3, SEG), jnp.int32),
            pltpu.VMEM((BLK,), jnp.int32),
            pltpu.VMEM((BLK,), jnp.int32),
            pltpu.VMEM((cap + BLK,), jnp.int32),
            pltpu.VMEM((cap + BLK,), jnp.int32),
            pltpu.VMEM((cap + BLK,), jnp.float32),
            pltpu.VMEM((BLK, D), jnp.float32),
            pltpu.VMEM((8, D), jnp.float32),
            pltpu.VMEM_SHARED((acc_rows, D), jnp.float32),
            pltpu.SemaphoreType.DMA,
            pltpu.SemaphoreType.DMA,
            pltpu.SemaphoreType.DMA,
        ],
        compiler_params=pltpu.CompilerParams(needs_layout_passes=False),
    )
    def spmm(edges_hbm, w_hbm, table_hbm, out_hbm,
             stage, src_v, dst_v, psrc, pldst, pw, rows_v, zero_v, acc,
             sem, sem_stage, sem_batch):
        cid = lax.axis_index("c")
        tid = lax.axis_index("s")
        dump_row = chunk_rows + tid
        del w_hbm

        def zrow(r, _):
            for v in range(D // 16):
                zero_v[r, pl.ds(v * 16, 16)] = jnp.zeros((16,), jnp.float32)
            return 0
        lax.fori_loop(0, 8, zrow, 0)

        def flush_block(b, _):
            for v in range(BLK // 16):
                sl = pl.ds(v * 16, 16)
                src_v[sl] = psrc[pl.ds(b * BLK + v * 16, 16)]
                dst_v[sl] = pldst[pl.ds(b * BLK + v * 16, 16)]
            pltpu.async_copy(table_hbm.at[src_v], rows_v, sem).wait()

            def scale(g, _):
                w16 = pw[pl.ds(b * BLK + g * 16, 16)]
                for l in range(16):
                    w = w16[l]
                    j = g * 16 + l
                    for v in range(D // 16):
                        sl = pl.ds(v * 16, 16)
                        rows_v[j, sl] = rows_v[j, sl] * w
                return 0
            lax.fori_loop(0, BLK // 16, scale, 0)
            pltpu.sync_copy(rows_v, acc.at[dst_v], add=True)
            return 0

        def chunk_body(c, _):
            cb = (cid * n_chunks_per_sc + c) * chunk_rows

            nz = acc_rows // (NT * 8)
            for z in range(nz):
                r0 = tid * (acc_rows // NT) + z * 8
                pltpu.async_copy(zero_v, acc.at[pl.ds(r0, 8)], sem_batch)
            for z in range(nz):
                pltpu.make_async_copy(zero_v, acc.at[pl.ds(0, 8)],
                                      sem_batch).wait()
            plsc.subcore_barrier()

            sbase = tid * stripe
            pltpu.async_copy(edges_hbm.at[:, pl.ds(sbase, SEG)], stage.at[0],
                             sem_stage)

            def seg_body(s, pend):
                par = s % 2
                pltpu.make_async_copy(edges_hbm.at[:, pl.ds(0, SEG)],
                                      stage.at[0], sem_stage).wait()

                @pl.when(s + 1 < nseg)
                def _():
                    pltpu.async_copy(
                        edges_hbm.at[:, pl.ds(sbase + (s + 1) * SEG, SEG)],
                        stage.at[1 - par], sem_stage)

                for blk in range(SEG // BLK):
                    for v in range(BLK // 16):
                        sl = pl.ds(blk * BLK + v * 16, 16)
                        l16 = stage[par, 1, sl] - cb
                        inr = (l16 >= 0) & (l16 < chunk_rows)
                        plsc.store_compressed(psrc.at[pl.ds(pend, 16)],
                                              stage[par, 0, sl], mask=inr)
                        plsc.store_compressed(pldst.at[pl.ds(pend, 16)],
                                              l16, mask=inr)
                        plsc.store_compressed(
                            pw.at[pl.ds(pend, 16)],
                            plsc.bitcast(stage[par, 2, sl], jnp.float32),
                            mask=inr)
                        pend = pend + plsc.all_reduce_population_count(inr)[0]

                    def overflow(p):
                        nfull = p >> 7
                        lax.fori_loop(0, nfull, flush_block, 0)
                        for v in range(BLK // 16):
                            sl = pl.ds(v * 16, 16)
                            off = pl.ds(nfull * BLK + v * 16, 16)
                            psrc[sl] = psrc[off]
                            pldst[sl] = pldst[off]
                            pw[sl] = pw[off]
                        return p & (BLK - 1)
                    pend = lax.cond(pend >= cap - BLK, overflow,
                                    lambda p: p, pend)
                return pend
            pend = lax.fori_loop(0, nseg, seg_body, jnp.int32(0))

            for v in range(BLK // 16):
                off = pl.ds(pend + v * 16, 16)
                psrc[off] = jnp.zeros((16,), jnp.int32)
                pldst[off] = jnp.full((16,), dump_row, jnp.int32)
                pw[off] = jnp.zeros((16,), jnp.float32)
            nflush = (pend + BLK - 1) >> 7
            lax.fori_loop(0, nflush, flush_block, 0)
            plsc.subcore_barrier()

            rpt = chunk_rows // NT
            for z in range(rpt // 32):
                r0 = tid * rpt + z * 32
                pltpu.async_copy(acc.at[pl.ds(r0, 32)],
                                 out_hbm.at[pl.ds(cb + r0, 32)], sem_batch)
            for z in range(rpt // 32):
                pltpu.make_async_copy(acc.at[pl.ds(0, 32)],
                                      out_hbm.at[pl.ds(cb, 32)],
                                      sem_batch).wait()
            plsc.subcore_barrier()
            return 0
        lax.fori_loop(0, n_chunks_per_sc, chunk_body, 0)

    return spmm


def _pack_edges(src, dst, w, n_pad):
    e = src.shape[0]
    pad = n_pad - e
    src = jnp.concatenate([src, jnp.zeros((pad,), jnp.int32)])
    dst = jnp.concatenate([dst, jnp.full((pad,), PAD_DST, jnp.int32)])
    w = jnp.concatenate([w, jnp.zeros((pad,), jnp.float32)])
    wbits = lax.bitcast_convert_type(w, jnp.int32)
    return jnp.stack([src, dst, wbits]), w


_E_PAD = 409600

_down_spmm = _make_spmm(_E_PAD, out_rows=10240, chunk_rows=5120,
                        acc_rows=5632, n_chunks_per_sc=1, pcap=152)
_up_spmm = _make_spmm(_E_PAD, out_rows=102400, chunk_rows=10240,
                      acc_rows=10752, n_chunks_per_sc=5, pcap=48)


def kernel(x, down_w, up_w, down_edge_index, up_edge_index):
    xl = x[:, -1]
    b, e, g, f = xl.shape
    x2d = xl.reshape(g, f)
    dpack, dw = _pack_edges(down_edge_index[0], down_edge_index[1],
                            down_w, _E_PAD)
    upack, uw = _pack_edges(up_edge_index[0], up_edge_index[1],
                            up_w, _E_PAD)
    down_table = _down_spmm(dpack, dw, x2d)
    up = _up_spmm(upack, uw, down_table)
    return up[:NUM_DATA].reshape(b, e, NUM_DATA, f)
